# baseline (speedup 1.0000x reference)
import jax
import jax.numpy as jnp
from jax.experimental import pallas as pl

_G = 16
_BN = 256


def _body(sqv_ref, phig_ref, um_ref, cvec_ref, kxz_ref, wub_ref,
          x_ref, d_ref, eu_ref, un_ref,
          big_ref, u_ref):
    f32 = jnp.float32
    L = um_ref.shape[0]
    K = sqv_ref.shape[1]

    def dot(a, b, prec=None):
        return jax.lax.dot_general(a, b, (((1,), (0,)), ((), ())),
                                   precision=prec,
                                   preferred_element_type=f32)

    hi = jax.lax.Precision.HIGHEST

    u = eu_ref[...] * sqv_ref[...]
    u_ref[...] = u
    ut = dot(u, wub_ref[...])

    um = um_ref[...]
    cvec = cvec_ref[...]
    kxz = kxz_ref[...]
    phig = phig_ref[...]

    lanes = d_ref.shape[1]
    n_pairs = lanes // L
    tail = lanes - n_pairs * L

    def step(sl, W, umW, cvecW, kxzW, utW, zlast):
        zl = dot(d_ref[:, sl], umW, hi)
        zrow = zl + jnp.tile(zlast, (1, W // K)) * cvecW
        big_ref[:, lanes + sl.start:lanes + sl.stop] = zrow
        xz = jnp.concatenate([x_ref[:, sl], zrow], axis=1)
        lin = dot(xz, kxzW) + utW
        big_ref[:, 2 * lanes + sl.start:2 * lanes + sl.stop] = lin
        m = jax.nn.sigmoid(lin)
        big_ref[:, 3 * lanes + sl.start:3 * lanes + sl.stop] = m
        big_ref[:, sl] = (un_ref[:, sl] < m).astype(f32)
        return zl

    zlast = jnp.zeros((d_ref.shape[0], K), f32)
    for p in range(n_pairs):
        sl = slice(p * L, (p + 1) * L)
        zl = step(sl, L, um, cvec, kxz, ut, zlast)
        zlast = zl[:, L - K:] + phig * zlast

    if tail:
        sl = slice(n_pairs * L, lanes)
        kxzT = jnp.concatenate([kxz[:tail, :tail],
                                kxz[L:L + tail, :tail]], axis=0)
        step(sl, tail, um[:tail, :tail], cvec[:, :tail], kxzT,
             ut[:, :tail], zlast)


def kernel(x, Wx, Wz, Wu, phi, var_u, eps_u, eps_d, unif):
    f32 = jnp.float32
    N, T, K = x.shape
    OUT = Wx.shape[1]
    L = _G * K

    phi_s = phi.astype(f32)[0]
    idx = jnp.arange(_G)
    expo = (idx[None, :] - idx[:, None]).astype(f32)
    tri = jnp.where(idx[None, :] >= idx[:, None],
                    phi_s ** jnp.maximum(expo, 0.0), 0.0)
    eyeK = jnp.eye(K, dtype=f32)
    um = jnp.kron(tri, eyeK)
    cpow = phi_s ** jnp.arange(1, _G + 1, dtype=f32)
    cvec = jnp.kron(cpow[None, :], jnp.ones((1, K), f32))
    eyeG = jnp.eye(_G, dtype=f32)
    wxb = jnp.kron(eyeG, Wx.astype(f32))
    wzb = jnp.kron(eyeG, Wz.astype(f32))
    wub = jnp.kron(jnp.ones((1, _G), f32), Wu.astype(f32))
    sqv = jnp.broadcast_to(jnp.sqrt(var_u.astype(f32)), (1, K))
    kxz = jnp.concatenate([wxb, wzb], axis=0)
    phig = jnp.broadcast_to(phi_s ** _G, (1, K))

    x3 = x.reshape(N, T * K)
    d3 = eps_d.reshape(N, T * K)
    un3 = unif.reshape(N, T * K)
    eu2 = eps_u.reshape(N, K)

    bs3 = pl.BlockSpec((_BN, T * K), lambda i: (i, 0))
    bs2 = pl.BlockSpec((_BN, K), lambda i: (i, 0))

    def bsw(shape):
        return pl.BlockSpec(shape, lambda i: tuple(0 for _ in shape))

    bsbig = pl.BlockSpec((_BN, 4 * T * K), lambda i: (i, 0))
    big4, u2 = pl.pallas_call(
        _body,
        grid=(N // _BN,),
        in_specs=[bsw((1, K)), bsw((1, K)), bsw((L, L)), bsw((1, L)),
                  bsw((2 * L, L)), bsw((K, L)), bs3, bs3, bs2, bs3],
        out_specs=(bsbig, bs2),
        out_shape=(jax.ShapeDtypeStruct((N, 4 * T * K), f32),
                   jax.ShapeDtypeStruct((N, K), f32)),
    )(sqv, phig, um, cvec, kxz, wub, x3, d3, eu2, un3)

    TK = T * K
    y = big4[:, :TK].reshape(N, T, OUT)
    z = big4[:, TK:2 * TK].reshape(N, T, OUT)
    u = u2.reshape(N, 1, K)
    linpar = big4[:, 2 * TK:3 * TK].reshape(N, T, OUT)
    mean = big4[:, 3 * TK:].reshape(N, T, OUT)
    return (y, z, u, linpar, mean)


# speedup vs baseline: 1.2743x; 1.2743x over previous
import jax
import jax.numpy as jnp
from jax.experimental import pallas as pl

_G = 16
_BN = 256


def _body(sqv_ref, phig_ref, um_ref, cvec_ref, kxz_ref, wub_ref,
          x_ref, d_ref, eu_ref, un_ref,
          y_ref, z_ref, u_ref, lin_ref, mean_ref):
    f32 = jnp.float32
    L = um_ref.shape[0]
    K = sqv_ref.shape[1]

    def dot(a, b, prec=None):
        return jax.lax.dot_general(a, b, (((1,), (0,)), ((), ())),
                                   precision=prec,
                                   preferred_element_type=f32)

    hi = jax.lax.Precision.HIGHEST

    u = eu_ref[...] * sqv_ref[...]
    u_ref[...] = u
    ut = dot(u, wub_ref[...])

    um = um_ref[...]
    cvec = cvec_ref[...]
    kxz = kxz_ref[...]
    phig = phig_ref[...]

    lanes = d_ref.shape[1]
    n_pairs = lanes // L
    tail = lanes - n_pairs * L

    def step(sl, W, umW, cvecW, kxzW, utW, zlast):
        zl = dot(d_ref[:, sl], umW, hi)
        zrow = zl + jnp.tile(zlast, (1, W // K)) * cvecW
        z_ref[:, sl] = zrow
        xz = jnp.concatenate([x_ref[:, sl], zrow], axis=1)
        lin = dot(xz, kxzW) + utW
        lin_ref[:, sl] = lin
        m = jax.nn.sigmoid(lin)
        mean_ref[:, sl] = m
        y_ref[:, sl] = (un_ref[:, sl] < m).astype(f32)
        return zl

    zlast = jnp.zeros((d_ref.shape[0], K), f32)
    for p in range(n_pairs):
        sl = slice(p * L, (p + 1) * L)
        zl = step(sl, L, um, cvec, kxz, ut, zlast)
        zlast = zl[:, L - K:] + phig * zlast

    if tail:
        sl = slice(n_pairs * L, lanes)
        kxzT = jnp.concatenate([kxz[:tail, :tail],
                                kxz[L:L + tail, :tail]], axis=0)
        step(sl, tail, um[:tail, :tail], cvec[:, :tail], kxzT,
             ut[:, :tail], zlast)


def kernel(x, Wx, Wz, Wu, phi, var_u, eps_u, eps_d, unif):
    f32 = jnp.float32
    N, T, K = x.shape
    OUT = Wx.shape[1]
    L = _G * K

    phi_s = phi.astype(f32)[0]
    idx = jnp.arange(_G)
    expo = (idx[None, :] - idx[:, None]).astype(f32)
    tri = jnp.where(idx[None, :] >= idx[:, None],
                    phi_s ** jnp.maximum(expo, 0.0), 0.0)
    eyeK = jnp.eye(K, dtype=f32)
    um = jnp.kron(tri, eyeK)
    cpow = phi_s ** jnp.arange(1, _G + 1, dtype=f32)
    cvec = jnp.kron(cpow[None, :], jnp.ones((1, K), f32))
    eyeG = jnp.eye(_G, dtype=f32)
    wxb = jnp.kron(eyeG, Wx.astype(f32))
    wzb = jnp.kron(eyeG, Wz.astype(f32))
    wub = jnp.kron(jnp.ones((1, _G), f32), Wu.astype(f32))
    sqv = jnp.broadcast_to(jnp.sqrt(var_u.astype(f32)), (1, K))
    kxz = jnp.concatenate([wxb, wzb], axis=0)
    phig = jnp.broadcast_to(phi_s ** _G, (1, K))

    x3 = x.reshape(N, T * K)
    d3 = eps_d.reshape(N, T * K)
    un3 = unif.reshape(N, T * K)
    eu2 = eps_u.reshape(N, K)

    bs3 = pl.BlockSpec((_BN, T * K), lambda i: (i, 0))
    bs2 = pl.BlockSpec((_BN, K), lambda i: (i, 0))

    def bsw(shape):
        return pl.BlockSpec(shape, lambda i: tuple(0 for _ in shape))

    big = jax.ShapeDtypeStruct((N, T * K), f32)
    y3, z3, u2, lin3, m3 = pl.pallas_call(
        _body,
        grid=(N // _BN,),
        in_specs=[bsw((1, K)), bsw((1, K)), bsw((L, L)), bsw((1, L)),
                  bsw((2 * L, L)), bsw((K, L)), bs3, bs3, bs2, bs3],
        out_specs=(bs3, bs3, bs2, bs3, bs3),
        out_shape=(big, big, jax.ShapeDtypeStruct((N, K), f32), big, big),
    )(sqv, phig, um, cvec, kxz, wub, x3, d3, eu2, un3)

    y = y3.reshape(N, T, OUT)
    z = z3.reshape(N, T, OUT)
    u = u2.reshape(N, 1, K)
    linpar = lin3.reshape(N, T, OUT)
    mean = m3.reshape(N, T, OUT)
    return (y, z, u, linpar, mean)
